# Initial kernel scaffold; baseline (speedup 1.0000x reference)
import jax
import jax.numpy as jnp
from jax.experimental import pallas as pl


def kernel(x, conv1_w, conv1_b, conv2_w, conv2_b, conv3_w, conv3_b, conv4_w, conv4_b, deconv4_w, deconv4_b, deconv3_w, deconv3_b, deconv2_w, deconv2_b, classifier_w, classifier_b):
    raise NotImplementedError("write your pallas kernel here")


# speedup vs baseline: 5.6909x; 5.6909x over previous
import functools

import jax
import jax.numpy as jnp
from jax.experimental import pallas as pl
from jax.experimental.pallas import tpu as pltpu


def _act(x, act):
    if act == "leaky":
        return jnp.where(x >= 0.0, x, 0.2 * x)
    if act == "sigmoid":
        return jnp.clip(pl.reciprocal(1.0 + jnp.exp(-x), approx=True), 0.0, 1.0)
    return x


def _conv_kernel(x_ref, w_ref, b_ref, o_ref, *, k, stride, Ho, Wo, act):
    x = x_ref[...]
    B, _, _, Cin = x.shape
    y = None
    for dy in range(k):
        for dx in range(k):
            sl = x[:, dy:dy + stride * Ho, dx:dx + stride * Wo, :]
            if stride > 1:
                sl = sl.reshape(B, Ho, stride, Wo, stride, Cin)[:, :, 0, :, 0, :]
            p = sl.reshape(B * Ho * Wo, Cin)
            t = dy * k + dx
            w_tap = w_ref[t * Cin:(t + 1) * Cin, :]
            d = jnp.dot(p, w_tap, preferred_element_type=jnp.float32)
            y = d if y is None else y + d
    y = _act(y + b_ref[...], act)
    o_ref[...] = y.reshape(B, Ho, Wo, -1).astype(o_ref.dtype)


def _conv(x, w, b, stride, pad, act, bgroup):
    N, H, W, Cin = x.shape
    Cout, _, k, _ = w.shape
    Ho = (H + 2 * pad - k) // stride + 1
    Wo = (W + 2 * pad - k) // stride + 1
    Hp = max(H + 2 * pad, k - 1 + stride * Ho)
    Wp = max(W + 2 * pad, k - 1 + stride * Wo)
    xp = jnp.pad(x, ((0, 0), (pad, Hp - H - pad), (pad, Wp - W - pad), (0, 0)))
    wmat = w.transpose(2, 3, 1, 0).reshape(k * k * Cin, Cout).astype(jnp.bfloat16)
    brow = b.astype(jnp.float32).reshape(1, Cout)
    out = pl.pallas_call(
        functools.partial(_conv_kernel, k=k, stride=stride, Ho=Ho, Wo=Wo, act=act),
        out_shape=jax.ShapeDtypeStruct((N, Ho, Wo, Cout), jnp.bfloat16),
        grid=(N // bgroup,),
        in_specs=[
            pl.BlockSpec((bgroup, Hp, Wp, Cin), lambda n: (n, 0, 0, 0)),
            pl.BlockSpec((k * k * Cin, Cout), lambda n: (0, 0)),
            pl.BlockSpec((1, Cout), lambda n: (0, 0)),
        ],
        out_specs=pl.BlockSpec((bgroup, Ho, Wo, Cout), lambda n: (n, 0, 0, 0)),
        compiler_params=pltpu.CompilerParams(
            dimension_semantics=("parallel",)),
    )(xp, wmat, brow)
    return out


def _mm_kernel(a_ref, w_ref, b_ref, o_ref, *, act):
    y = jnp.dot(a_ref[...], w_ref[...], preferred_element_type=jnp.float32)
    o_ref[...] = _act(y + b_ref[...], act).astype(o_ref.dtype)


def _conv1(x, w, b):
    N, H, W, _ = x.shape
    Cout = w.shape[0]
    Ho, Wo = H // 2, W // 2
    xp = jnp.pad(x[..., 0], ((0, 0), (1, 1), (1, 1)))
    cols = jnp.stack(
        [xp[:, dy:dy + 2 * Ho:2, dx:dx + 2 * Wo:2]
         for dy in range(3) for dx in range(3)], axis=-1)
    a = cols.reshape(N * Ho * Wo, 9).astype(jnp.bfloat16)
    wmat = w.transpose(2, 3, 1, 0).reshape(9, Cout).astype(jnp.bfloat16)
    brow = b.astype(jnp.float32).reshape(1, Cout)
    tm = 4096
    M = N * Ho * Wo
    out = pl.pallas_call(
        functools.partial(_mm_kernel, act="leaky"),
        out_shape=jax.ShapeDtypeStruct((M, Cout), jnp.bfloat16),
        grid=(M // tm,),
        in_specs=[
            pl.BlockSpec((tm, 9), lambda i: (i, 0)),
            pl.BlockSpec((9, Cout), lambda i: (0, 0)),
            pl.BlockSpec((1, Cout), lambda i: (0, 0)),
        ],
        out_specs=pl.BlockSpec((tm, Cout), lambda i: (i, 0)),
        compiler_params=pltpu.CompilerParams(
            dimension_semantics=("parallel",)),
    )(a, wmat, brow)
    return out.reshape(N, Ho, Wo, Cout)


def _up_kernel(x_ref, uh_ref, uw_ref, o_ref, *, act):
    t = jnp.dot(uh_ref[...], x_ref[0], preferred_element_type=jnp.float32)
    y = jnp.dot(t.astype(jnp.bfloat16), uw_ref[...],
                preferred_element_type=jnp.float32)
    o_ref[0] = _act(y, act).astype(o_ref.dtype)


def _up_matrix(n_in, n_out):
    scale = n_in / n_out
    dst = jnp.arange(n_out, dtype=jnp.float32)
    src = jnp.clip((dst + 0.5) * scale - 0.5, 0.0)
    x0 = jnp.minimum(jnp.floor(src).astype(jnp.int32), n_in - 1)
    x1 = jnp.minimum(x0 + 1, n_in - 1)
    lam = src - x0.astype(jnp.float32)
    U = jnp.zeros((n_out, n_in), jnp.float32)
    idx = jnp.arange(n_out)
    U = U.at[idx, x0].add(1.0 - lam)
    U = U.at[idx, x1].add(lam)
    return U


def _upsample(x, scale, bgroup, act="none", out_dtype=jnp.bfloat16):
    N, H, W, C = x.shape
    Ho, Wo = H * scale, W * scale
    Uh = _up_matrix(H, Ho)
    UhB = jnp.kron(jnp.eye(bgroup, dtype=jnp.float32), Uh).astype(jnp.bfloat16)
    Uw = _up_matrix(W, Wo)
    UwC = jnp.kron(Uw.T, jnp.eye(C, dtype=jnp.float32)).astype(jnp.bfloat16)
    x2d = x.reshape(N // bgroup, bgroup * H, W * C)
    out = pl.pallas_call(
        functools.partial(_up_kernel, act=act),
        out_shape=jax.ShapeDtypeStruct((N // bgroup, bgroup * Ho, Wo * C),
                                       out_dtype),
        grid=(N // bgroup,),
        in_specs=[
            pl.BlockSpec((1, bgroup * H, W * C), lambda n: (n, 0, 0)),
            pl.BlockSpec((bgroup * Ho, bgroup * H), lambda n: (0, 0)),
            pl.BlockSpec((W * C, Wo * C), lambda n: (0, 0)),
        ],
        out_specs=pl.BlockSpec((1, bgroup * Ho, Wo * C), lambda n: (n, 0, 0)),
        compiler_params=pltpu.CompilerParams(
            dimension_semantics=("parallel",)),
    )(x2d, UhB, UwC)
    return out.reshape(N, Ho, Wo, C)


@jax.jit
def _forward(x_nchw, params):
    (c1w, c1b, c2w, c2b, c3w, c3b, c4w, c4b,
     d4w, d4b, d3w, d3b, d2w, d2b, clw, clb) = params
    x = x_nchw.transpose(0, 2, 3, 1)
    x = _conv1(x, c1w, c1b)
    x = _conv(x, c2w, c2b, 2, 1, "leaky", bgroup=4)
    x = _conv(x, c3w, c3b, 2, 1, "leaky", bgroup=8)
    x = _conv(x, c4w, c4b, 2, 1, "leaky", bgroup=16)
    x = _upsample(x, 2, bgroup=8)
    x = _conv(x, d4w, d4b, 1, 1, "leaky", bgroup=4)
    x = _upsample(x, 2, bgroup=8)
    x = _conv(x, d3w, d3b, 1, 1, "leaky", bgroup=2)
    x = _upsample(x, 2, bgroup=4)
    x = _conv(x, d2w, d2b, 1, 1, "leaky", bgroup=2)
    x = _conv(x, clw, clb, 2, 1, "none", bgroup=1)
    x = _upsample(x, 4, bgroup=16, act="sigmoid",
                  out_dtype=jnp.float32)
    return x.transpose(0, 3, 1, 2)


def kernel(x, conv1_w, conv1_b, conv2_w, conv2_b, conv3_w, conv3_b,
           conv4_w, conv4_b, deconv4_w, deconv4_b, deconv3_w, deconv3_b,
           deconv2_w, deconv2_b, classifier_w, classifier_b):
    params = (conv1_w, conv1_b, conv2_w, conv2_b, conv3_w, conv3_b,
              conv4_w, conv4_b, deconv4_w, deconv4_b, deconv3_w, deconv3_b,
              deconv2_w, deconv2_b, classifier_w, classifier_b)
    return _forward(x, params)
